# baseline (speedup 1.0000x reference)
import functools

import jax
import jax.numpy as jnp
from jax import lax
from jax.experimental import pallas as pl
from jax.experimental.pallas import tpu as pltpu
from jax.experimental.pallas import tpu_sc as plsc

NR_CATE = 26
BATCH = 16384
NR_FEAT = 128

NC = 2
NS = 16
L = 16
NW = NC * NS
BPW = BATCH // NW
CHUNK = 128
NCH = BPW // CHUNK
SENT = -1


def kernel(x, label):
    mesh = plsc.VectorSubcoreMesh(core_axis_name="c", subcore_axis_name="s")

    @functools.partial(
        pl.kernel,
        mesh=mesh,
        out_type=jax.ShapeDtypeStruct((BATCH, 1, NR_FEAT), jnp.float32),
        scratch_types=[
            pltpu.VMEM((BPW,), jnp.int32),
            pltpu.VMEM((NR_CATE, NCH, CHUNK), jnp.int32),
            pltpu.VMEM((BPW, 1, NR_FEAT), jnp.float32),
            pltpu.SemaphoreType.DMA,
        ],
    )
    def k(x_hbm, label_hbm, out_hbm, label_v, idx3, rows_v, sem):
        wid = lax.axis_index("s") * NC + lax.axis_index("c")
        base = wid * BPW
        pltpu.sync_copy(label_hbm.at[pl.ds(base, BPW)], label_v)
        lane = lax.iota(jnp.int32, L)

        def build(kk, carry):
            for j in range(CHUNK // L):
                off = kk * CHUNK + j * L
                lab = label_v[pl.ds(off, L)]
                bidx = base + off + lane
                for c in range(NR_CATE):
                    idx3[c, kk, pl.ds(j * L, L)] = jnp.where(
                        lab == c, bidx, SENT
                    )
            return carry

        lax.fori_loop(0, NCH, build, 0)

        copies = []
        for c in range(NR_CATE):
            for kk in range(NCH):
                copies.append(
                    pltpu.async_copy(
                        x_hbm.at[
                            plsc.Indices(idx3.at[c, kk], ignored_value=SENT),
                            pl.ds(c, 1),
                        ],
                        rows_v.at[pl.ds(kk * CHUNK, CHUNK)],
                        sem,
                    )
                )
        for cp in copies:
            cp.wait()
        pltpu.sync_copy(rows_v, out_hbm.at[pl.ds(base, BPW)])

    return k(x, label).reshape(BATCH, NR_FEAT)


# speedup vs baseline: 15.6637x; 15.6637x over previous
import functools

import jax
import jax.numpy as jnp
from jax import lax
from jax.experimental import pallas as pl
from jax.experimental.pallas import tpu as pltpu
from jax.experimental.pallas import tpu_sc as plsc

NR_CATE = 26
CATE_PAD = 32
BATCH = 16384
NR_FEAT = 128

NC = 2
NS = 16
L = 16
NW = NC * NS
BPW = BATCH // NW
CHUNK = 128
NCH = BPW // CHUNK


def kernel(x, label):
    mesh = plsc.VectorSubcoreMesh(core_axis_name="c", subcore_axis_name="s")

    @functools.partial(
        pl.kernel,
        mesh=mesh,
        out_type=jax.ShapeDtypeStruct((BATCH, NR_FEAT), jnp.float32),
        scratch_types=[
            pltpu.VMEM((BPW,), jnp.int32),
            pltpu.VMEM((NCH, CHUNK), jnp.int32),
            pltpu.VMEM((BPW, NR_FEAT), jnp.float32),
            pltpu.SemaphoreType.DMA,
        ],
    )
    def k(x_hbm, label_hbm, out_hbm, label_v, idx_v, rows_v, sem):
        wid = lax.axis_index("s") * NC + lax.axis_index("c")
        base = wid * BPW
        pltpu.sync_copy(label_hbm.at[pl.ds(base, BPW)], label_v)
        lane = lax.iota(jnp.int32, L)
        for c in range(NCH):
            for j in range(CHUNK // L):
                off = c * CHUNK + j * L
                lab = label_v[pl.ds(off, L)]
                idx_v[c, pl.ds(j * L, L)] = (base + off + lane) * CATE_PAD + lab
        table = x_hbm.at[0]
        copies = [
            pltpu.async_copy(
                table.at[idx_v.at[c]], rows_v.at[pl.ds(c * CHUNK, CHUNK)], sem
            )
            for c in range(NCH)
        ]
        for cp in copies:
            cp.wait()
        pltpu.sync_copy(rows_v, out_hbm.at[pl.ds(base, BPW)])

    return k(x, label)


# speedup vs baseline: 15.6798x; 1.0010x over previous
import functools

import jax
import jax.numpy as jnp
from jax import lax
from jax.experimental import pallas as pl
from jax.experimental.pallas import tpu as pltpu
from jax.experimental.pallas import tpu_sc as plsc

NR_CATE = 26
CATE_PAD = 32
BATCH = 16384
NR_FEAT = 128

NC = 2
NS = 16
L = 16
NW = NC * NS
BPW = BATCH // NW
CHUNK = 128
NCH = BPW // CHUNK


def kernel(x, label):
    mesh = plsc.VectorSubcoreMesh(core_axis_name="c", subcore_axis_name="s")

    @functools.partial(
        pl.kernel,
        mesh=mesh,
        out_type=jax.ShapeDtypeStruct((BATCH, NR_FEAT), jnp.float32),
        scratch_types=[
            pltpu.VMEM((BPW,), jnp.int32),
            pltpu.VMEM((NCH, CHUNK), jnp.int32),
            pltpu.VMEM((BPW, NR_FEAT), jnp.float32),
            pltpu.SemaphoreType.DMA,
        ],
    )
    def k(x_hbm, label_hbm, out_hbm, label_v, idx_v, rows_v, sem):
        wid = lax.axis_index("s") * NC + lax.axis_index("c")
        base = wid * BPW
        pltpu.sync_copy(label_hbm.at[pl.ds(base, BPW)], label_v)
        lane = lax.iota(jnp.int32, L)
        table = x_hbm.at[0]
        copies = []
        for g in range(BPW // L):
            off = g * L
            lab = label_v[pl.ds(off, L)]
            idxv = (base + off + lane) * CATE_PAD + lab
            copies.append(
                pltpu.async_copy(
                    table.at[idxv], rows_v.at[pl.ds(off, L)], sem
                )
            )
        for cp in copies:
            cp.wait()
        pltpu.sync_copy(rows_v, out_hbm.at[pl.ds(base, BPW)])

    return k(x, label)
